# baseline (speedup 1.0000x reference)
import functools

import jax
import jax.numpy as jnp
from jax import lax
from jax.experimental import pallas as pl
from jax.experimental.pallas import tpu as pltpu
from jax.experimental.pallas import tpu_sc as plsc

_POSE_NUM = 100000
_EMBED_DIM = 6
_BATCH = 16384

_NC = 2
_NS = 16
_NW = _NC * _NS
_B_PER_W = _BATCH // _NW

_mesh = plsc.VectorSubcoreMesh(core_axis_name="c", subcore_axis_name="s")


@functools.partial(
    pl.kernel,
    mesh=_mesh,
    out_type=jax.ShapeDtypeStruct((_EMBED_DIM, _BATCH), jnp.float32),
    scratch_types=[
        pltpu.VMEM((_B_PER_W,), jnp.int32),
        pltpu.VMEM((_EMBED_DIM, _B_PER_W), jnp.float32),
        pltpu.SemaphoreType.DMA,
    ],
    compiler_params=pltpu.CompilerParams(
        use_tc_tiling_on_sc=False,
        needs_layout_passes=False,
        skip_device_barrier=True,
    ),
)
def _sc_gather(idx_hbm, table_t_hbm, out_t_hbm, idx_v, cols_v, sem):
    cid = lax.axis_index("c")
    sid = lax.axis_index("s")
    wid = sid * _NC + cid
    base = wid * _B_PER_W

    pltpu.sync_copy(idx_hbm.at[pl.ds(base, _B_PER_W)], idx_v)

    chunks = []
    for d in range(_EMBED_DIM):
        chunks.append(
            pltpu.async_copy(
                table_t_hbm.at[d].at[idx_v],
                cols_v.at[d],
                sem,
            )
        )
    for c in chunks:
        c.wait()
    pltpu.sync_copy(cols_v, out_t_hbm.at[:, pl.ds(base, _B_PER_W)])


def kernel(indices, table):
    out_t = _sc_gather(indices.astype(jnp.int32), table.T)
    return out_t.T


# speedup vs baseline: 1.0037x; 1.0037x over previous
import functools

import jax
import jax.numpy as jnp
from jax import lax
from jax.experimental import pallas as pl
from jax.experimental.pallas import tpu as pltpu
from jax.experimental.pallas import tpu_sc as plsc

_POSE_NUM = 100000
_EMBED_DIM = 6
_BATCH = 16384

_NC = 2
_NS = 16
_NW = _NC * _NS
_B_PER_W = _BATCH // _NW

_mesh = plsc.VectorSubcoreMesh(core_axis_name="c", subcore_axis_name="s")


@functools.partial(
    pl.kernel,
    mesh=_mesh,
    out_type=jax.ShapeDtypeStruct((_EMBED_DIM, _BATCH), jnp.float32),
    scratch_types=[
        pltpu.VMEM((_B_PER_W,), jnp.int32),
        pltpu.VMEM((_EMBED_DIM, _B_PER_W), jnp.float32),
        pltpu.SemaphoreType.DMA,
    ],
    compiler_params=pltpu.CompilerParams(
        use_tc_tiling_on_sc=False, needs_layout_passes=False
    ),
)
def _sc_gather(idx_hbm, table_t_hbm, out_t_hbm, idx_v, cols_v, sem):
    cid = lax.axis_index("c")
    sid = lax.axis_index("s")
    wid = sid * _NC + cid
    base = wid * _B_PER_W

    pltpu.sync_copy(idx_hbm.at[pl.ds(base, _B_PER_W)], idx_v)

    chunks = []
    for d in range(_EMBED_DIM):
        chunks.append(
            pltpu.async_copy(
                table_t_hbm.at[d].at[idx_v],
                cols_v.at[d],
                sem,
            )
        )
    for c in chunks:
        c.wait()
    pltpu.sync_copy(cols_v, out_t_hbm.at[:, pl.ds(base, _B_PER_W)])


def kernel(indices, table):
    out_t = _sc_gather(indices.astype(jnp.int32), table.T)
    return out_t.T
